# Initial kernel scaffold; baseline (speedup 1.0000x reference)
import jax
import jax.numpy as jnp
from jax.experimental import pallas as pl


def kernel(h, edge_index, W_edge, b_edge, W_node, b_node):
    raise NotImplementedError("write your pallas kernel here")


# speedup vs baseline: 25.6271x; 25.6271x over previous
import functools

import jax
import jax.numpy as jnp
from jax import lax
from jax.experimental import pallas as pl
from jax.experimental.pallas import tpu as pltpu
from jax.experimental.pallas import tpu_sc as plsc

N = 10000
H = 128
E = 320000
NUM_CORES = 2
NUM_SUBCORES = 16
NUM_TILES = NUM_CORES * NUM_SUBCORES
E_TILE = E // NUM_TILES
ROWS = 80
E_TILE_PAD = ROWS * 128
PAD = E_TILE_PAD - E_TILE


def _node_scores_tc(h, w3, b3):
    def body(h_ref, w_ref, b_ref, o_ref):
        s = lax.dot_general(
            w_ref[...], h_ref[...], (((1,), (1,)), ((), ())),
            preferred_element_type=jnp.float32)
        o_ref[...] = s + b_ref[...]

    return pl.pallas_call(
        body,
        out_shape=jax.ShapeDtypeStruct((3, N), jnp.float32),
    )(h, w3, b3)


def _edge_accumulate_sc(s3n, src3, dst3, zeros):
    mesh = plsc.VectorSubcoreMesh(core_axis_name="c", subcore_axis_name="s")

    @functools.partial(
        pl.kernel,
        out_type=jax.ShapeDtypeStruct((NUM_CORES, N), jnp.float32),
        mesh=mesh,
        compiler_params=pltpu.CompilerParams(needs_layout_passes=False),
        scratch_types=[
            pltpu.VMEM((ROWS, 128), jnp.int32),
            pltpu.VMEM((ROWS, 128), jnp.int32),
            pltpu.VMEM((ROWS, 128), jnp.float32),
            pltpu.VMEM((N,), jnp.float32),
            pltpu.VMEM((N,), jnp.float32),
            pltpu.VMEM_SHARED((N,), jnp.float32),
        ],
    )
    def k(s_hbm, src_hbm, dst_hbm, z_hbm, out_hbm,
          src_v, dst_v, vals_v, s1_v, s2_v, acc_sh):
        c = lax.axis_index("c")
        s = lax.axis_index("s")
        wid = c * NUM_SUBCORES + s

        pltpu.sync_copy(src_hbm.at[wid], src_v)
        pltpu.sync_copy(dst_hbm.at[wid], dst_v)
        pltpu.sync_copy(s_hbm.at[0], s1_v)
        pltpu.sync_copy(s_hbm.at[1], s2_v)

        @pl.when(s == 0)
        def _():
            pltpu.sync_copy(z_hbm, acc_sh)

        def row(j, carry):
            def col(kk, carry2):
                si = src_v[j, pl.ds(kk * 16, 16)]
                di = dst_v[j, pl.ds(kk * 16, 16)]
                g = plsc.load_gather(s1_v, [si]) + plsc.load_gather(s2_v, [di])
                eid = j * 128 + kk * 16 + lax.iota(jnp.int32, 16)
                g = jnp.where(eid < E_TILE, g, jnp.zeros_like(g))
                vals_v[j, pl.ds(kk * 16, 16)] = g
                return carry2

            return lax.fori_loop(0, 8, col, carry)

        lax.fori_loop(0, ROWS, row, 0)

        plsc.subcore_barrier()

        def srow(j, carry):
            pltpu.sync_copy(vals_v.at[j], acc_sh.at[dst_v.at[j]], add=True)
            return carry

        lax.fori_loop(0, ROWS, srow, 0)
        plsc.subcore_barrier()

        @pl.when(s == 0)
        def _():
            pltpu.sync_copy(acc_sh, out_hbm.at[c])

    return k(s3n, src3, dst3, zeros)


def _combine_softmax_tc(parts, s3n):
    def body(p_ref, s_ref, o_ref):
        combined = p_ref[0:1, :] + p_ref[1:2, :] + s_ref[2:3, :]
        m = jnp.max(combined)
        e = jnp.exp(combined - m)
        o_ref[...] = e / jnp.sum(e)

    return pl.pallas_call(
        body,
        out_shape=jax.ShapeDtypeStruct((1, N), jnp.float32),
    )(parts, s3n)


def kernel(h, edge_index, W_edge, b_edge, W_node, b_node):
    h = h.astype(jnp.float32)
    src = edge_index[0].astype(jnp.int32).reshape(NUM_TILES, E_TILE)
    dst = edge_index[1].astype(jnp.int32).reshape(NUM_TILES, E_TILE)
    pad_src = jnp.zeros((NUM_TILES, PAD), jnp.int32)
    pad_dst = jnp.broadcast_to(
        (jnp.arange(PAD, dtype=jnp.int32) * 41) % N, (NUM_TILES, PAD))
    src3 = jnp.concatenate([src, pad_src], axis=1).reshape(NUM_TILES, ROWS, 128)
    dst3 = jnp.concatenate([dst, pad_dst], axis=1).reshape(NUM_TILES, ROWS, 128)

    w3 = jnp.stack([W_edge[:H], W_edge[H:], W_node], axis=0)
    b3 = jnp.stack(
        [jnp.zeros_like(b_edge), b_edge, b_node]).reshape(3, 1).astype(jnp.float32)

    s3n = _node_scores_tc(h, w3, b3)
    zeros = jnp.zeros((N,), jnp.float32)
    parts = _edge_accumulate_sc(s3n, src3, dst3, zeros)
    out = _combine_softmax_tc(parts, s3n)
    return out.reshape(N)


# speedup vs baseline: 37.2102x; 1.4520x over previous
import functools

import jax
import jax.numpy as jnp
from jax import lax
from jax.experimental import pallas as pl
from jax.experimental.pallas import tpu as pltpu
from jax.experimental.pallas import tpu_sc as plsc

N = 10000
H = 128
E = 320000
NUM_CORES = 2
NUM_SUBCORES = 16
NUM_TILES = NUM_CORES * NUM_SUBCORES
E_TILE = E // NUM_TILES
UNROLL = 5
CHUNKS = E_TILE // (16 * UNROLL)


def _node_scores_tc(h, w3, b3):
    def body(h_ref, w_ref, b_ref, o_ref):
        s = lax.dot_general(
            w_ref[...], h_ref[...], (((1,), (1,)), ((), ())),
            preferred_element_type=jnp.float32)
        o_ref[...] = s + b_ref[...]

    return pl.pallas_call(
        body,
        out_shape=jax.ShapeDtypeStruct((3, N), jnp.float32),
    )(h, w3, b3)


def _edge_accumulate_sc(s3n, edge_index, zeros):
    mesh = plsc.VectorSubcoreMesh(core_axis_name="c", subcore_axis_name="s")

    @functools.partial(
        pl.kernel,
        out_type=jax.ShapeDtypeStruct((NUM_CORES, N), jnp.float32),
        mesh=mesh,
        compiler_params=pltpu.CompilerParams(needs_layout_passes=False),
        scratch_types=[
            pltpu.VMEM((E_TILE,), jnp.int32),
            pltpu.VMEM((E_TILE,), jnp.int32),
            pltpu.VMEM((E_TILE,), jnp.float32),
            pltpu.VMEM((N,), jnp.float32),
            pltpu.VMEM((N,), jnp.float32),
            pltpu.VMEM_SHARED((N,), jnp.float32),
        ],
    )
    def k(s_hbm, ei_hbm, z_hbm, out_hbm,
          src_v, dst_v, vals_v, s1_v, s2_v, acc_sh):
        c = lax.axis_index("c")
        s = lax.axis_index("s")
        wid = c * NUM_SUBCORES + s
        base = wid * E_TILE

        pltpu.sync_copy(ei_hbm.at[pl.ds(base, E_TILE)], src_v)
        pltpu.sync_copy(ei_hbm.at[pl.ds(E + base, E_TILE)], dst_v)
        pltpu.sync_copy(s_hbm.at[0], s1_v)
        pltpu.sync_copy(s_hbm.at[1], s2_v)

        @pl.when(s == 0)
        def _():
            pltpu.sync_copy(z_hbm, acc_sh)

        plsc.subcore_barrier()

        def chunk(i, carry):
            b0 = i * (16 * UNROLL)
            for u in range(UNROLL):
                si = src_v[pl.ds(b0 + u * 16, 16)]
                di = dst_v[pl.ds(b0 + u * 16, 16)]
                g = plsc.load_gather(s1_v, [si]) + plsc.load_gather(s2_v, [di])
                vals_v[pl.ds(b0 + u * 16, 16)] = g
            return carry

        lax.fori_loop(0, CHUNKS, chunk, 0)

        pltpu.sync_copy(vals_v, acc_sh.at[dst_v], add=True)
        plsc.subcore_barrier()

        @pl.when(s == 0)
        def _():
            pltpu.sync_copy(acc_sh, out_hbm.at[c])

    return k(s3n, edge_index, zeros)


def _combine_softmax_tc(parts, s3n):
    def body(p_ref, s_ref, o_ref):
        combined = p_ref[0:1, :] + p_ref[1:2, :] + s_ref[2:3, :]
        m = jnp.max(combined)
        e = jnp.exp(combined - m)
        o_ref[...] = e / jnp.sum(e)

    return pl.pallas_call(
        body,
        out_shape=jax.ShapeDtypeStruct((1, N), jnp.float32),
    )(parts, s3n)


def kernel(h, edge_index, W_edge, b_edge, W_node, b_node):
    h = h.astype(jnp.float32)
    ei = edge_index.astype(jnp.int32).reshape(2 * E)

    w3 = jnp.stack([W_edge[:H], W_edge[H:], W_node], axis=0)
    b3 = jnp.stack(
        [jnp.zeros_like(b_edge), b_edge, b_node]).reshape(3, 1).astype(jnp.float32)

    s3n = _node_scores_tc(h, w3, b3)
    zeros = jnp.zeros((N,), jnp.float32)
    parts = _edge_accumulate_sc(s3n, ei, zeros)
    out = _combine_softmax_tc(parts, s3n)
    return out.reshape(N)
